# baseline (speedup 1.0000x reference)
import functools

import jax
import jax.numpy as jnp
from jax import lax
from jax.experimental import pallas as pl
from jax.experimental.pallas import tpu as pltpu
from jax.experimental.pallas import tpu_sc as plsc

_NUM_NODES = 1000000
_DIM = 32
_BATCH = 16384

_NC = 2
_NS = 16
_NW = _NC * _NS
_PAIRS_PER_W = _BATCH // _NW
_CHUNK = 128
_NCHUNK = _PAIRS_PER_W // _CHUNK
_BLOCKS = _PAIRS_PER_W // 16


def _body(n1_hbm, n2_hbm, tt_hbm, out_hbm, idxa_v, idxb_v, colsa_v, colsb_v,
          out_v, sem):
    wid = lax.axis_index("s") * _NC + lax.axis_index("c")

    pltpu.sync_copy(n1_hbm.at[pl.ds(wid * _NCHUNK, _NCHUNK), :], idxa_v)
    pltpu.sync_copy(n2_hbm.at[pl.ds(wid * _NCHUNK, _NCHUNK), :], idxb_v)

    copies = []
    for d in range(_DIM):
        view = tt_hbm.at[d]
        for j in range(_NCHUNK):
            for ids, cols in ((idxa_v, colsa_v), (idxb_v, colsb_v)):
                copies.append(
                    pltpu.async_copy(
                        view.at[ids.at[j]],
                        cols.at[d, pl.ds(j * _CHUNK, _CHUNK)],
                        sem,
                    )
                )
    for c in copies:
        c.wait()

    def block(b, _):
        sl = pl.ds(b * 16, 16)
        acc = jnp.zeros((16,), jnp.float32)
        for d in range(_DIM):
            diff = colsa_v[d, sl] - colsb_v[d, sl]
            acc = acc + diff * diff
        out_v[sl] = acc
        return _

    lax.fori_loop(0, _BLOCKS, block, None)

    pltpu.sync_copy(out_v, out_hbm.at[pl.ds(wid * _PAIRS_PER_W, _PAIRS_PER_W)])


@jax.jit
def kernel(inputs, embedding_table):
    ids = inputs.astype(jnp.int32)
    n1 = ids[:, 0].reshape(_NW * _NCHUNK, _CHUNK)
    n2 = ids[:, 1].reshape(_NW * _NCHUNK, _CHUNK)
    table_t = embedding_table.T
    run = functools.partial(
        pl.kernel,
        mesh=plsc.VectorSubcoreMesh(core_axis_name="c", subcore_axis_name="s"),
        out_type=jax.ShapeDtypeStruct((_BATCH,), jnp.float32),
        compiler_params=pltpu.CompilerParams(
            needs_layout_passes=False, use_tc_tiling_on_sc=False
        ),
        scratch_types=[
            pltpu.VMEM((_NCHUNK, _CHUNK), jnp.int32),
            pltpu.VMEM((_NCHUNK, _CHUNK), jnp.int32),
            pltpu.VMEM((_DIM, _PAIRS_PER_W), jnp.float32),
            pltpu.VMEM((_DIM, _PAIRS_PER_W), jnp.float32),
            pltpu.VMEM((_PAIRS_PER_W,), jnp.float32),
            pltpu.SemaphoreType.DMA,
        ],
    )(_body)
    return run(n1, n2, table_t)


# speedup vs baseline: 4.8437x; 4.8437x over previous
import functools

import jax
import jax.numpy as jnp
from jax import lax
from jax.experimental import pallas as pl
from jax.experimental.pallas import tpu as pltpu
from jax.experimental.pallas import tpu_sc as plsc

_NUM_NODES = 1000000
_DIM = 32
_BATCH = 16384

_NC = 2
_NS = 16
_NW = _NC * _NS
_PAIRS_PER_W = _BATCH // _NW
_ROWS_PER_W = 2 * _PAIRS_PER_W
_CHUNK = 128
_NCHUNK = _ROWS_PER_W // _CHUNK
_PASS_ROWS = 512
_PASS_PAIRS = 256
_PASS_BLOCKS = _PASS_PAIRS // 16


def _body(ids_hbm, t4_hbm, out_hbm, idx_v, off_v, rows_v, out_v, sem):
    wid = lax.axis_index("s") * _NC + lax.axis_index("c")

    pltpu.sync_copy(ids_hbm.at[pl.ds(wid * _NCHUNK, _NCHUNK), :], idx_v)

    for j in range(_NCHUNK):
        for k in range(_CHUNK // 16):
            sl = pl.ds(k * 16, 16)
            r = idx_v[j, sl]
            off_v[j, sl] = (r & 3) << 5
            idx_v[j, sl] = r >> 2

    lanes = lax.broadcasted_iota(jnp.int32, (16,), 0)

    for p in range(2):
        copies = []
        for c in range(4):
            copies.append(
                pltpu.async_copy(
                    t4_hbm.at[idx_v.at[p * 4 + c]],
                    rows_v.at[pl.ds(c * _CHUNK, _CHUNK), :],
                    sem,
                )
            )
        for cp in copies:
            cp.wait()

        def block(b, _):
            row_a = 32 * b + 2 * lanes
            row_b = row_a + 1
            ga = p * _PASS_ROWS + row_a
            gb = ga + 1
            off_a = plsc.load_gather(off_v, [ga >> 7, ga & 127])
            off_b = plsc.load_gather(off_v, [gb >> 7, gb & 127])
            acc = jnp.zeros((16,), jnp.float32)
            for d in range(_DIM):
                a = plsc.load_gather(rows_v, [row_a, off_a + d])
                bb = plsc.load_gather(rows_v, [row_b, off_b + d])
                diff = a - bb
                acc = acc + diff * diff
            out_v[pl.ds(p * _PASS_PAIRS + b * 16, 16)] = acc
            return _

        lax.fori_loop(0, _PASS_BLOCKS, block, None)

    pltpu.sync_copy(out_v, out_hbm.at[pl.ds(wid * _PAIRS_PER_W, _PAIRS_PER_W)])


@jax.jit
def kernel(inputs, embedding_table):
    ids2d = inputs.astype(jnp.int32).reshape(_NW * _NCHUNK, _CHUNK)
    t4 = embedding_table.reshape(_NUM_NODES // 4, 4 * _DIM)
    run = functools.partial(
        pl.kernel,
        mesh=plsc.VectorSubcoreMesh(core_axis_name="c", subcore_axis_name="s"),
        out_type=jax.ShapeDtypeStruct((_BATCH,), jnp.float32),
        compiler_params=pltpu.CompilerParams(
            needs_layout_passes=False, use_tc_tiling_on_sc=False
        ),
        scratch_types=[
            pltpu.VMEM((_NCHUNK, _CHUNK), jnp.int32),
            pltpu.VMEM((_NCHUNK, _CHUNK), jnp.int32),
            pltpu.VMEM((_PASS_ROWS, 4 * _DIM), jnp.float32),
            pltpu.VMEM((_PAIRS_PER_W,), jnp.float32),
            pltpu.SemaphoreType.DMA,
        ],
    )(_body)
    return run(ids2d, t4)
